# baseline (speedup 1.0000x reference)
import functools

import jax
import jax.numpy as jnp
import numpy as np
from jax import lax
from jax.experimental import pallas as pl
from jax.experimental.pallas import tpu as pltpu
from jax.experimental.pallas import tpu_sc as plsc

N = 10000
E = 320000
NFEAT = 128
NHID = 128
NCLASS = 64
K = 32

NC = 2
NS = 16
NW = NC * NS
EPW = E // NW
NPAD = 10240
ROWS_PER_SUB = NPAD // NS

RB = 2000
GRID = N // RB


NBUF = 5
LA = 3


def _make_spmm(D, CHUNK):
  NCHUNK = EPW // CHUNK
  mesh = plsc.VectorSubcoreMesh(core_axis_name="c", subcore_axis_name="s")

  @functools.partial(
      pl.kernel,
      out_type=jax.ShapeDtypeStruct((NC, NPAD, D), jnp.float32),
      mesh=mesh,
      compiler_params=pltpu.CompilerParams(use_tc_tiling_on_sc=False),
      scratch_types=[
          pltpu.VMEM_SHARED((NPAD, D), jnp.float32),
          pltpu.VMEM((EPW,), jnp.int32),
          pltpu.VMEM((NCHUNK, CHUNK), jnp.int32),
          pltpu.VMEM((EPW,), jnp.float32),
          [pltpu.VMEM((CHUNK, D // 2), jnp.int32) for _ in range(NBUF)],
          [pltpu.VMEM((CHUNK, D), jnp.float32) for _ in range(NBUF)],
          [pltpu.SemaphoreType.DMA for _ in range(NBUF)],
          [pltpu.SemaphoreType.DMA for _ in range(NBUF)],
      ],
  )
  def spmm(src_hbm, dst_hbm, ev_hbm, dense_hbm, out_hbm,
           acc, src_v, dst_v, ev_v, rows, msgs, gsem, ssem):
    cid = lax.axis_index("c")
    sid = lax.axis_index("s")
    wid = sid * NC + cid

    pltpu.async_copy(src_hbm.at[wid], src_v, gsem[0])
    pltpu.async_copy(dst_hbm.at[wid], dst_v, gsem[1])
    pltpu.async_copy(ev_hbm.at[wid], ev_v, gsem[2])

    @pl.loop(0, CHUNK)
    def _zero_row(i):
      for f in range(D // 16):
        msgs[0][i, pl.ds(f * 16, 16)] = jnp.zeros((16,), jnp.float32)

    row0 = sid * ROWS_PER_SUB

    @pl.loop(0, ROWS_PER_SUB // CHUNK)
    def _zero_acc(j):
      pltpu.sync_copy(msgs[0], acc.at[pl.ds(row0 + j * CHUNK, CHUNK)])

    pltpu.make_async_copy(src_hbm.at[wid], src_v, gsem[0]).wait()
    pltpu.make_async_copy(dst_hbm.at[wid], dst_v, gsem[1]).wait()
    pltpu.make_async_copy(ev_hbm.at[wid], ev_v, gsem[2]).wait()
    plsc.subcore_barrier()

    def gather_start(g, b):
      pltpu.async_copy(
          dense_hbm.at[src_v.at[pl.ds(g * CHUNK, CHUNK)]], rows[b], gsem[b])

    def gather_wait(b):
      pltpu.make_async_copy(dense_hbm.at[pl.ds(0, CHUNK)], rows[b],
                            gsem[b]).wait()

    def scatter_wait(b):
      pltpu.make_async_copy(msgs[b], acc.at[pl.ds(0, CHUNK)], ssem[b]).wait()

    for j in range(LA):
      gather_start(j, j)

    @pl.loop(0, NCHUNK // NBUF)
    def _block(t):
      for j in range(NBUF):
        g = t * NBUF + j
        gather_wait(j)

        for gg in range(CHUNK // 16):
          ev16 = ev_v[pl.ds(g * CHUNK + gg * 16, 16)]
          for l in range(16):
            s = ev16[l]
            e = gg * 16 + l
            for q in range(D // 32):
              w = rows[j][e, pl.ds(q * 16, 16)]
              lo = lax.bitcast_convert_type(w << 16, jnp.float32)
              hi = lax.bitcast_convert_type(w & jnp.int32(-65536), jnp.float32)
              msgs[j][e, pl.ds(q * 32, 16)] = lo * s
              msgs[j][e, pl.ds(q * 32 + 16, 16)] = hi * s

        pltpu.async_copy(msgs[j], acc.at[dst_v.at[g]], ssem[j], add=True)

        nb = (j + LA) % NBUF

        @pl.when(jnp.logical_and(g + LA < NCHUNK, g >= NBUF - LA))
        def _():
          scatter_wait(nb)

        @pl.when(g + LA < NCHUNK)
        def _():
          gather_start(g + LA, nb)

    for j in range(NBUF):
      scatter_wait(j)
    plsc.subcore_barrier()
    pltpu.sync_copy(acc.at[pl.ds(row0, ROWS_PER_SUB)],
                    out_hbm.at[cid, pl.ds(row0, ROWS_PER_SUB)])

  return spmm


CHUNK_HID = 16
CHUNK_CLS = 80
_spmm_hid = _make_spmm(NHID, CHUNK_HID)
_spmm_cls = _make_spmm(NCLASS, CHUNK_CLS)


def _dense1_body(x_ref, w1_ref, v_ref, sup_ref, tilda_ref):
  i = pl.program_id(0)
  s = jnp.dot(x_ref[...], w1_ref[...], preferred_element_type=jnp.float32)
  sup_ref[...] = s
  t = lax.dot_general(v_ref[...], s, (((0,), (0,)), ((), ())),
                      preferred_element_type=jnp.float32)

  @pl.when(i == 0)
  def _():
    tilda_ref[...] = t

  @pl.when(i > 0)
  def _():
    tilda_ref[...] += t


def _dense1(x, w1, v):
  return pl.pallas_call(
      _dense1_body,
      grid=(GRID,),
      in_specs=[
          pl.BlockSpec((RB, NFEAT), lambda i: (i, 0)),
          pl.BlockSpec((NFEAT, NHID), lambda i: (0, 0)),
          pl.BlockSpec((RB, K), lambda i: (i, 0)),
      ],
      out_specs=[
          pl.BlockSpec((RB, NHID), lambda i: (i, 0)),
          pl.BlockSpec((K, NHID), lambda i: (0, 0)),
      ],
      out_shape=[
          jax.ShapeDtypeStruct((N, NHID), jnp.float32),
          jax.ShapeDtypeStruct((K, NHID), jnp.float32),
      ],
  )(x, w1, v)


def _dense2_body(agg_ref, v_ref, tilda_ref, delta_ref, b1_ref, w2_ref,
                 pinv_ref, sup2_ref, tilda2_ref):
  i = pl.program_id(0)
  r = jnp.dot(v_ref[...], delta_ref[...] * tilda_ref[...],
              preferred_element_type=jnp.float32)
  agg = jnp.dot(agg_ref[0] + agg_ref[1], pinv_ref[...],
                preferred_element_type=jnp.float32)
  h = agg + r + b1_ref[...]
  h = jnp.maximum(h, 0.0)
  s2 = jnp.dot(h, w2_ref[...], preferred_element_type=jnp.float32)
  sup2_ref[...] = s2
  t2 = lax.dot_general(v_ref[...], s2, (((0,), (0,)), ((), ())),
                       preferred_element_type=jnp.float32)

  @pl.when(i == 0)
  def _():
    tilda2_ref[...] = t2

  @pl.when(i > 0)
  def _():
    tilda2_ref[...] += t2


def _dense2(agg, v, tilda, delta, b1, w2, pinv):
  return pl.pallas_call(
      _dense2_body,
      grid=(GRID,),
      in_specs=[
          pl.BlockSpec((NC, RB, NHID), lambda i: (0, i, 0)),
          pl.BlockSpec((RB, K), lambda i: (i, 0)),
          pl.BlockSpec((K, NHID), lambda i: (0, 0)),
          pl.BlockSpec((K, 1), lambda i: (0, 0)),
          pl.BlockSpec((1, NHID), lambda i: (0, 0)),
          pl.BlockSpec((NHID, NCLASS), lambda i: (0, 0)),
          pl.BlockSpec((NHID, NHID), lambda i: (0, 0)),
      ],
      out_specs=[
          pl.BlockSpec((RB, NCLASS), lambda i: (i, 0)),
          pl.BlockSpec((K, NCLASS), lambda i: (0, 0)),
      ],
      out_shape=[
          jax.ShapeDtypeStruct((N, NCLASS), jnp.float32),
          jax.ShapeDtypeStruct((K, NCLASS), jnp.float32),
      ],
  )(agg, v, tilda, delta, b1, w2, pinv)


def _dense3_body(agg_ref, v_ref, tilda2_ref, delta_ref, b2_ref, pinv_ref,
                 out_ref):
  r = jnp.dot(v_ref[...], delta_ref[...] * tilda2_ref[...],
              preferred_element_type=jnp.float32)
  o = jnp.dot(agg_ref[0] + agg_ref[1], pinv_ref[...],
              preferred_element_type=jnp.float32) + r + b2_ref[...]
  m = jnp.max(o, axis=1, keepdims=True)
  e = o - m
  lse = jnp.log(jnp.sum(jnp.exp(e), axis=1, keepdims=True))
  out_ref[...] = e - lse


def _dense3(agg, v, tilda2, delta, b2, pinv):
  return pl.pallas_call(
      _dense3_body,
      grid=(GRID,),
      in_specs=[
          pl.BlockSpec((NC, RB, NCLASS), lambda i: (0, i, 0)),
          pl.BlockSpec((RB, K), lambda i: (i, 0)),
          pl.BlockSpec((K, NCLASS), lambda i: (0, 0)),
          pl.BlockSpec((K, 1), lambda i: (0, 0)),
          pl.BlockSpec((1, NCLASS), lambda i: (0, 0)),
          pl.BlockSpec((NCLASS, NCLASS), lambda i: (0, 0)),
      ],
      out_specs=pl.BlockSpec((RB, NCLASS), lambda i: (i, 0)),
      out_shape=jax.ShapeDtypeStruct((N, NCLASS), jnp.float32),
  )(agg, v, tilda2, delta, b2, pinv)


def _unperm_matrix(D):
  sigma = np.empty(D, dtype=np.int64)
  for j in range(D):
    q, r = divmod(j, 32)
    sigma[j] = 32 * q + 2 * r if r < 16 else 32 * q + 2 * (r - 16) + 1
  m = np.zeros((D, D), dtype=np.float32)
  m[np.arange(D), sigma] = 1.0
  return m


_PINV_HID = _unperm_matrix(NHID)
_PINV_CLS = _unperm_matrix(NCLASS)


def kernel(x, edge_index, edge_vals, eigvec_mat, gc1_weight, gc1_bias,
           gc2_weight, gc2_bias, delta):
  dst_h = edge_index[0].reshape(NW, EPW // CHUNK_HID, CHUNK_HID)
  dst_c = edge_index[0].reshape(NW, EPW // CHUNK_CLS, CHUNK_CLS)
  src = edge_index[1].reshape(NW, EPW)
  ev = edge_vals.reshape(NW, EPW)
  delta2d = delta[:, None]
  b1 = gc1_bias[None, :]
  b2 = gc2_bias[None, :]

  support, tilda = _dense1(x, gc1_weight, eigvec_mat)
  table1 = lax.bitcast_convert_type(
      support.astype(jnp.bfloat16).reshape(N, NHID // 2, 2), jnp.int32)
  part1 = _spmm_hid(src, dst_h, ev, table1)
  support2, tilda2 = _dense2(part1, eigvec_mat, tilda, delta2d, b1,
                             gc2_weight, _PINV_HID)
  table2 = lax.bitcast_convert_type(
      support2.astype(jnp.bfloat16).reshape(N, NCLASS // 2, 2), jnp.int32)
  part2 = _spmm_cls(src, dst_c, ev, table2)
  return _dense3(part2, eigvec_mat, tilda2, delta2d, b2, _PINV_CLS)


# speedup vs baseline: 1.0474x; 1.0474x over previous
import functools

import jax
import jax.numpy as jnp
from jax import lax
from jax.experimental import pallas as pl
from jax.experimental.pallas import tpu as pltpu
from jax.experimental.pallas import tpu_sc as plsc

N = 10000
E = 320000
NFEAT = 128
NHID = 128
NCLASS = 64
K = 32

NC = 2
NS = 16
NW = NC * NS
EPW = E // NW
NPAD = 10240
ROWS_PER_SUB = NPAD // NS

RB = 2000
GRID = N // RB


NBUF = 5
LA = 3


def _make_spmm(D, CHUNK):
  NCHUNK = EPW // CHUNK
  mesh = plsc.VectorSubcoreMesh(core_axis_name="c", subcore_axis_name="s")

  @functools.partial(
      pl.kernel,
      out_type=jax.ShapeDtypeStruct((NC, NPAD, D), jnp.float32),
      mesh=mesh,
      compiler_params=pltpu.CompilerParams(use_tc_tiling_on_sc=False),
      scratch_types=[
          pltpu.VMEM_SHARED((NPAD, D), jnp.float32),
          pltpu.VMEM((EPW,), jnp.int32),
          pltpu.VMEM((NCHUNK, CHUNK), jnp.int32),
          pltpu.VMEM((EPW,), jnp.float32),
          [pltpu.VMEM((CHUNK, D), jnp.float32) for _ in range(NBUF)],
          [pltpu.SemaphoreType.DMA for _ in range(NBUF)],
          [pltpu.SemaphoreType.DMA for _ in range(NBUF)],
      ],
  )
  def spmm(src_hbm, dst_hbm, ev_hbm, dense_hbm, out_hbm,
           acc, src_v, dst_v, ev_v, rows, gsem, ssem):
    cid = lax.axis_index("c")
    sid = lax.axis_index("s")
    wid = sid * NC + cid

    pltpu.async_copy(src_hbm.at[wid], src_v, gsem[0])
    pltpu.async_copy(dst_hbm.at[wid], dst_v, gsem[1])
    pltpu.async_copy(ev_hbm.at[wid], ev_v, gsem[2])

    @pl.loop(0, CHUNK)
    def _zero_row(i):
      for f in range(D // 16):
        rows[0][i, pl.ds(f * 16, 16)] = jnp.zeros((16,), jnp.float32)

    row0 = sid * ROWS_PER_SUB

    @pl.loop(0, ROWS_PER_SUB // CHUNK)
    def _zero_acc(j):
      pltpu.sync_copy(rows[0], acc.at[pl.ds(row0 + j * CHUNK, CHUNK)])

    pltpu.make_async_copy(src_hbm.at[wid], src_v, gsem[0]).wait()
    pltpu.make_async_copy(dst_hbm.at[wid], dst_v, gsem[1]).wait()
    pltpu.make_async_copy(ev_hbm.at[wid], ev_v, gsem[2]).wait()
    plsc.subcore_barrier()

    def gather_start(g, b):
      pltpu.async_copy(
          dense_hbm.at[src_v.at[pl.ds(g * CHUNK, CHUNK)]], rows[b], gsem[b])

    def gather_wait(b):
      pltpu.make_async_copy(dense_hbm.at[pl.ds(0, CHUNK)], rows[b],
                            gsem[b]).wait()

    def scatter_wait(b):
      pltpu.make_async_copy(rows[b], acc.at[pl.ds(0, CHUNK)], ssem[b]).wait()

    for j in range(LA):
      gather_start(j, j)

    @pl.loop(0, NCHUNK // NBUF)
    def _block(t):
      for j in range(NBUF):
        g = t * NBUF + j
        gather_wait(j)

        for gg in range(CHUNK // 16):
          ev16 = ev_v[pl.ds(g * CHUNK + gg * 16, 16)]
          for l in range(16):
            s = ev16[l]
            e = gg * 16 + l
            for f in range(D // 16):
              sl = pl.ds(f * 16, 16)
              rows[j][e, sl] = rows[j][e, sl] * s

        pltpu.async_copy(rows[j], acc.at[dst_v.at[g]], ssem[j], add=True)

        nb = (j + LA) % NBUF

        @pl.when(jnp.logical_and(g + LA < NCHUNK, g >= NBUF - LA))
        def _():
          scatter_wait(nb)

        @pl.when(g + LA < NCHUNK)
        def _():
          gather_start(g + LA, nb)

    for j in range(NBUF):
      scatter_wait(j)
    plsc.subcore_barrier()
    pltpu.sync_copy(acc.at[pl.ds(row0, ROWS_PER_SUB)],
                    out_hbm.at[cid, pl.ds(row0, ROWS_PER_SUB)])

  return spmm


CHUNK_HID = 16
CHUNK_CLS = 80
_spmm_hid = _make_spmm(NHID, CHUNK_HID)
_spmm_cls = _make_spmm(NCLASS, CHUNK_CLS)


def _dense1_body(x_ref, w1_ref, v_ref, sup_ref, tilda_ref):
  i = pl.program_id(0)
  s = jnp.dot(x_ref[...], w1_ref[...], preferred_element_type=jnp.float32)
  sup_ref[...] = s
  t = lax.dot_general(v_ref[...], s, (((0,), (0,)), ((), ())),
                      preferred_element_type=jnp.float32)

  @pl.when(i == 0)
  def _():
    tilda_ref[...] = t

  @pl.when(i > 0)
  def _():
    tilda_ref[...] += t


def _dense1(x, w1, v):
  return pl.pallas_call(
      _dense1_body,
      grid=(GRID,),
      in_specs=[
          pl.BlockSpec((RB, NFEAT), lambda i: (i, 0)),
          pl.BlockSpec((NFEAT, NHID), lambda i: (0, 0)),
          pl.BlockSpec((RB, K), lambda i: (i, 0)),
      ],
      out_specs=[
          pl.BlockSpec((RB, NHID), lambda i: (i, 0)),
          pl.BlockSpec((K, NHID), lambda i: (0, 0)),
      ],
      out_shape=[
          jax.ShapeDtypeStruct((N, NHID), jnp.float32),
          jax.ShapeDtypeStruct((K, NHID), jnp.float32),
      ],
  )(x, w1, v)


def _dense2_body(agg_ref, v_ref, tilda_ref, delta_ref, b1_ref, w2_ref,
                 sup2_ref, tilda2_ref):
  i = pl.program_id(0)
  r = jnp.dot(v_ref[...], delta_ref[...] * tilda_ref[...],
              preferred_element_type=jnp.float32)
  h = agg_ref[0] + agg_ref[1] + r + b1_ref[...]
  h = jnp.maximum(h, 0.0)
  s2 = jnp.dot(h, w2_ref[...], preferred_element_type=jnp.float32)
  sup2_ref[...] = s2
  t2 = lax.dot_general(v_ref[...], s2, (((0,), (0,)), ((), ())),
                       preferred_element_type=jnp.float32)

  @pl.when(i == 0)
  def _():
    tilda2_ref[...] = t2

  @pl.when(i > 0)
  def _():
    tilda2_ref[...] += t2


def _dense2(agg, v, tilda, delta, b1, w2):
  return pl.pallas_call(
      _dense2_body,
      grid=(GRID,),
      in_specs=[
          pl.BlockSpec((NC, RB, NHID), lambda i: (0, i, 0)),
          pl.BlockSpec((RB, K), lambda i: (i, 0)),
          pl.BlockSpec((K, NHID), lambda i: (0, 0)),
          pl.BlockSpec((K, 1), lambda i: (0, 0)),
          pl.BlockSpec((1, NHID), lambda i: (0, 0)),
          pl.BlockSpec((NHID, NCLASS), lambda i: (0, 0)),
      ],
      out_specs=[
          pl.BlockSpec((RB, NCLASS), lambda i: (i, 0)),
          pl.BlockSpec((K, NCLASS), lambda i: (0, 0)),
      ],
      out_shape=[
          jax.ShapeDtypeStruct((N, NCLASS), jnp.float32),
          jax.ShapeDtypeStruct((K, NCLASS), jnp.float32),
      ],
  )(agg, v, tilda, delta, b1, w2)


def _dense3_body(agg_ref, v_ref, tilda2_ref, delta_ref, b2_ref, out_ref):
  r = jnp.dot(v_ref[...], delta_ref[...] * tilda2_ref[...],
              preferred_element_type=jnp.float32)
  o = agg_ref[0] + agg_ref[1] + r + b2_ref[...]
  m = jnp.max(o, axis=1, keepdims=True)
  e = o - m
  lse = jnp.log(jnp.sum(jnp.exp(e), axis=1, keepdims=True))
  out_ref[...] = e - lse


def _dense3(agg, v, tilda2, delta, b2):
  return pl.pallas_call(
      _dense3_body,
      grid=(GRID,),
      in_specs=[
          pl.BlockSpec((NC, RB, NCLASS), lambda i: (0, i, 0)),
          pl.BlockSpec((RB, K), lambda i: (i, 0)),
          pl.BlockSpec((K, NCLASS), lambda i: (0, 0)),
          pl.BlockSpec((K, 1), lambda i: (0, 0)),
          pl.BlockSpec((1, NCLASS), lambda i: (0, 0)),
      ],
      out_specs=pl.BlockSpec((RB, NCLASS), lambda i: (i, 0)),
      out_shape=jax.ShapeDtypeStruct((N, NCLASS), jnp.float32),
  )(agg, v, tilda2, delta, b2)


def kernel(x, edge_index, edge_vals, eigvec_mat, gc1_weight, gc1_bias,
           gc2_weight, gc2_bias, delta):
  dst_h = edge_index[0].reshape(NW, EPW // CHUNK_HID, CHUNK_HID)
  dst_c = edge_index[0].reshape(NW, EPW // CHUNK_CLS, CHUNK_CLS)
  src = edge_index[1].reshape(NW, EPW)
  ev = edge_vals.reshape(NW, EPW)
  delta2d = delta[:, None]
  b1 = gc1_bias[None, :]
  b2 = gc2_bias[None, :]

  support, tilda = _dense1(x, gc1_weight, eigvec_mat)
  part1 = _spmm_hid(src, dst_h, ev, support)
  support2, tilda2 = _dense2(part1, eigvec_mat, tilda, delta2d, b1,
                             gc2_weight)
  part2 = _spmm_cls(src, dst_c, ev, support2)
  return _dense3(part2, eigvec_mat, tilda2, delta2d, b2)


# speedup vs baseline: 1.0510x; 1.0034x over previous
import functools

import jax
import jax.numpy as jnp
from jax import lax
from jax.experimental import pallas as pl
from jax.experimental.pallas import tpu as pltpu
from jax.experimental.pallas import tpu_sc as plsc

N = 10000
E = 320000
NFEAT = 128
NHID = 128
NCLASS = 64
K = 32

NC = 2
NS = 16
NW = NC * NS
EPW = E // NW
NPAD = 10240
ROWS_PER_SUB = NPAD // NS

RB = 2000
GRID = N // RB


NBUF = 5
LA = 3


def _make_spmm(D, CHUNK):
  NCHUNK = EPW // CHUNK
  mesh = plsc.VectorSubcoreMesh(core_axis_name="c", subcore_axis_name="s")

  @functools.partial(
      pl.kernel,
      out_type=jax.ShapeDtypeStruct((NC, NPAD, D), jnp.float32),
      mesh=mesh,
      compiler_params=pltpu.CompilerParams(use_tc_tiling_on_sc=False),
      scratch_types=[
          pltpu.VMEM_SHARED((NPAD, D), jnp.float32),
          pltpu.VMEM((EPW,), jnp.int32),
          pltpu.VMEM((NCHUNK, CHUNK), jnp.int32),
          pltpu.VMEM((EPW,), jnp.float32),
          [pltpu.VMEM((CHUNK, D), jnp.float32) for _ in range(NBUF)],
          [pltpu.SemaphoreType.DMA for _ in range(NBUF)],
          [pltpu.SemaphoreType.DMA for _ in range(NBUF)],
      ],
  )
  def spmm(src_hbm, dst_hbm, ev_hbm, dense_hbm, out_hbm,
           acc, src_v, dst_v, ev_v, rows, gsem, ssem):
    cid = lax.axis_index("c")
    sid = lax.axis_index("s")
    wid = sid * NC + cid

    pltpu.async_copy(src_hbm.at[wid], src_v, gsem[0])
    pltpu.async_copy(dst_hbm.at[wid], dst_v, gsem[1])
    pltpu.async_copy(ev_hbm.at[wid], ev_v, gsem[2])

    @pl.loop(0, CHUNK)
    def _zero_row(i):
      for f in range(D // 16):
        rows[0][i, pl.ds(f * 16, 16)] = jnp.zeros((16,), jnp.float32)

    row0 = sid * ROWS_PER_SUB

    for j in range(ROWS_PER_SUB // CHUNK):
      pltpu.async_copy(rows[0], acc.at[pl.ds(row0 + j * CHUNK, CHUNK)],
                       ssem[j % NBUF])
    for j in range(ROWS_PER_SUB // CHUNK):
      pltpu.make_async_copy(rows[0], acc.at[pl.ds(0, CHUNK)],
                            ssem[j % NBUF]).wait()

    pltpu.make_async_copy(src_hbm.at[wid], src_v, gsem[0]).wait()
    pltpu.make_async_copy(dst_hbm.at[wid], dst_v, gsem[1]).wait()
    pltpu.make_async_copy(ev_hbm.at[wid], ev_v, gsem[2]).wait()
    plsc.subcore_barrier()

    def gather_start(g, b):
      pltpu.async_copy(
          dense_hbm.at[src_v.at[pl.ds(g * CHUNK, CHUNK)]], rows[b], gsem[b])

    def gather_wait(b):
      pltpu.make_async_copy(dense_hbm.at[pl.ds(0, CHUNK)], rows[b],
                            gsem[b]).wait()

    def scatter_wait(b):
      pltpu.make_async_copy(rows[b], acc.at[pl.ds(0, CHUNK)], ssem[b]).wait()

    for j in range(LA):
      gather_start(j, j)

    @pl.loop(0, NCHUNK // NBUF)
    def _block(t):
      for j in range(NBUF):
        g = t * NBUF + j
        gather_wait(j)

        for gg in range(CHUNK // 16):
          ev16 = ev_v[pl.ds(g * CHUNK + gg * 16, 16)]
          for l in range(16):
            s = ev16[l]
            e = gg * 16 + l
            for f in range(D // 16):
              sl = pl.ds(f * 16, 16)
              rows[j][e, sl] = rows[j][e, sl] * s

        pltpu.async_copy(rows[j], acc.at[dst_v.at[g]], ssem[j], add=True)

        nb = (j + LA) % NBUF

        @pl.when(jnp.logical_and(g + LA < NCHUNK, g >= NBUF - LA))
        def _():
          scatter_wait(nb)

        @pl.when(g + LA < NCHUNK)
        def _():
          gather_start(g + LA, nb)

    for j in range(NBUF):
      scatter_wait(j)
    plsc.subcore_barrier()
    pltpu.sync_copy(acc.at[pl.ds(row0, ROWS_PER_SUB)],
                    out_hbm.at[cid, pl.ds(row0, ROWS_PER_SUB)])

  return spmm


CHUNK_HID = 16
CHUNK_CLS = 80
_spmm_hid = _make_spmm(NHID, CHUNK_HID)
_spmm_cls = _make_spmm(NCLASS, CHUNK_CLS)


def _dense1_body(x_ref, w1_ref, v_ref, sup_ref, tilda_ref):
  i = pl.program_id(0)
  s = jnp.dot(x_ref[...], w1_ref[...], preferred_element_type=jnp.float32)
  sup_ref[...] = s
  t = lax.dot_general(v_ref[...], s, (((0,), (0,)), ((), ())),
                      preferred_element_type=jnp.float32)

  @pl.when(i == 0)
  def _():
    tilda_ref[...] = t

  @pl.when(i > 0)
  def _():
    tilda_ref[...] += t


def _dense1(x, w1, v):
  return pl.pallas_call(
      _dense1_body,
      grid=(GRID,),
      in_specs=[
          pl.BlockSpec((RB, NFEAT), lambda i: (i, 0)),
          pl.BlockSpec((NFEAT, NHID), lambda i: (0, 0)),
          pl.BlockSpec((RB, K), lambda i: (i, 0)),
      ],
      out_specs=[
          pl.BlockSpec((RB, NHID), lambda i: (i, 0)),
          pl.BlockSpec((K, NHID), lambda i: (0, 0)),
      ],
      out_shape=[
          jax.ShapeDtypeStruct((N, NHID), jnp.float32),
          jax.ShapeDtypeStruct((K, NHID), jnp.float32),
      ],
  )(x, w1, v)


def _dense2_body(agg_ref, v_ref, tilda_ref, delta_ref, b1_ref, w2_ref,
                 sup2_ref, tilda2_ref):
  i = pl.program_id(0)
  r = jnp.dot(v_ref[...], delta_ref[...] * tilda_ref[...],
              preferred_element_type=jnp.float32)
  h = agg_ref[0] + agg_ref[1] + r + b1_ref[...]
  h = jnp.maximum(h, 0.0)
  s2 = jnp.dot(h, w2_ref[...], preferred_element_type=jnp.float32)
  sup2_ref[...] = s2
  t2 = lax.dot_general(v_ref[...], s2, (((0,), (0,)), ((), ())),
                       preferred_element_type=jnp.float32)

  @pl.when(i == 0)
  def _():
    tilda2_ref[...] = t2

  @pl.when(i > 0)
  def _():
    tilda2_ref[...] += t2


def _dense2(agg, v, tilda, delta, b1, w2):
  return pl.pallas_call(
      _dense2_body,
      grid=(GRID,),
      in_specs=[
          pl.BlockSpec((NC, RB, NHID), lambda i: (0, i, 0)),
          pl.BlockSpec((RB, K), lambda i: (i, 0)),
          pl.BlockSpec((K, NHID), lambda i: (0, 0)),
          pl.BlockSpec((K, 1), lambda i: (0, 0)),
          pl.BlockSpec((1, NHID), lambda i: (0, 0)),
          pl.BlockSpec((NHID, NCLASS), lambda i: (0, 0)),
      ],
      out_specs=[
          pl.BlockSpec((RB, NCLASS), lambda i: (i, 0)),
          pl.BlockSpec((K, NCLASS), lambda i: (0, 0)),
      ],
      out_shape=[
          jax.ShapeDtypeStruct((N, NCLASS), jnp.float32),
          jax.ShapeDtypeStruct((K, NCLASS), jnp.float32),
      ],
  )(agg, v, tilda, delta, b1, w2)


def _dense3_body(agg_ref, v_ref, tilda2_ref, delta_ref, b2_ref, out_ref):
  r = jnp.dot(v_ref[...], delta_ref[...] * tilda2_ref[...],
              preferred_element_type=jnp.float32)
  o = agg_ref[0] + agg_ref[1] + r + b2_ref[...]
  m = jnp.max(o, axis=1, keepdims=True)
  e = o - m
  lse = jnp.log(jnp.sum(jnp.exp(e), axis=1, keepdims=True))
  out_ref[...] = e - lse


def _dense3(agg, v, tilda2, delta, b2):
  return pl.pallas_call(
      _dense3_body,
      grid=(GRID,),
      in_specs=[
          pl.BlockSpec((NC, RB, NCLASS), lambda i: (0, i, 0)),
          pl.BlockSpec((RB, K), lambda i: (i, 0)),
          pl.BlockSpec((K, NCLASS), lambda i: (0, 0)),
          pl.BlockSpec((K, 1), lambda i: (0, 0)),
          pl.BlockSpec((1, NCLASS), lambda i: (0, 0)),
      ],
      out_specs=pl.BlockSpec((RB, NCLASS), lambda i: (i, 0)),
      out_shape=jax.ShapeDtypeStruct((N, NCLASS), jnp.float32),
  )(agg, v, tilda2, delta, b2)


def kernel(x, edge_index, edge_vals, eigvec_mat, gc1_weight, gc1_bias,
           gc2_weight, gc2_bias, delta):
  dst_h = edge_index[0].reshape(NW, EPW // CHUNK_HID, CHUNK_HID)
  dst_c = edge_index[0].reshape(NW, EPW // CHUNK_CLS, CHUNK_CLS)
  src = edge_index[1].reshape(NW, EPW)
  ev = edge_vals.reshape(NW, EPW)
  delta2d = delta[:, None]
  b1 = gc1_bias[None, :]
  b2 = gc2_bias[None, :]

  support, tilda = _dense1(x, gc1_weight, eigvec_mat)
  part1 = _spmm_hid(src, dst_h, ev, support)
  support2, tilda2 = _dense2(part1, eigvec_mat, tilda, delta2d, b1,
                             gc2_weight)
  part2 = _spmm_cls(src, dst_c, ev, support2)
  return _dense3(part2, eigvec_mat, tilda2, delta2d, b2)


# speedup vs baseline: 1.2035x; 1.1451x over previous
import functools

import jax
import jax.numpy as jnp
from jax import lax
from jax.experimental import pallas as pl
from jax.experimental.pallas import tpu as pltpu
from jax.experimental.pallas import tpu_sc as plsc

N = 10000
E = 320000
NFEAT = 128
NHID = 128
NCLASS = 64
K = 32

NC = 2
NS = 16
NW = NC * NS
EPW = E // NW
NPAD = 10240
ROWS_PER_SUB = NPAD // NS

RB = 2000
GRID = N // RB


NBUF = 5
LA = 4


def _make_spmm(D, CHUNK):
  NCHUNK = EPW // CHUNK
  mesh = plsc.VectorSubcoreMesh(core_axis_name="c", subcore_axis_name="s")

  @functools.partial(
      pl.kernel,
      out_type=jax.ShapeDtypeStruct((NC, NPAD, D), jnp.float32),
      mesh=mesh,
      compiler_params=pltpu.CompilerParams(use_tc_tiling_on_sc=False),
      scratch_types=[
          pltpu.VMEM_SHARED((NPAD, D), jnp.float32),
          pltpu.VMEM((EPW,), jnp.int32),
          pltpu.VMEM((NCHUNK, CHUNK), jnp.int32),
          pltpu.VMEM((EPW,), jnp.float32),
          [pltpu.VMEM((CHUNK, D), jnp.float32) for _ in range(NBUF)],
          [pltpu.SemaphoreType.DMA for _ in range(NBUF)],
          [pltpu.SemaphoreType.DMA for _ in range(NBUF)],
      ],
  )
  def spmm(src_hbm, dst_hbm, ev_hbm, dense_hbm, out_hbm,
           acc, src_v, dst_v, ev_v, rows, gsem, ssem):
    cid = lax.axis_index("c")
    sid = lax.axis_index("s")
    wid = sid * NC + cid

    pltpu.async_copy(src_hbm.at[wid], src_v, gsem[0])
    pltpu.async_copy(dst_hbm.at[wid], dst_v, gsem[1])
    pltpu.async_copy(ev_hbm.at[wid], ev_v, gsem[2])

    @pl.loop(0, CHUNK)
    def _zero_row(i):
      for f in range(D // 16):
        rows[0][i, pl.ds(f * 16, 16)] = jnp.zeros((16,), jnp.float32)

    row0 = sid * ROWS_PER_SUB

    for j in range(ROWS_PER_SUB // CHUNK):
      pltpu.async_copy(rows[0], acc.at[pl.ds(row0 + j * CHUNK, CHUNK)],
                       ssem[j % NBUF])
    for j in range(ROWS_PER_SUB // CHUNK):
      pltpu.make_async_copy(rows[0], acc.at[pl.ds(0, CHUNK)],
                            ssem[j % NBUF]).wait()

    pltpu.make_async_copy(src_hbm.at[wid], src_v, gsem[0]).wait()
    pltpu.make_async_copy(dst_hbm.at[wid], dst_v, gsem[1]).wait()
    pltpu.make_async_copy(ev_hbm.at[wid], ev_v, gsem[2]).wait()
    plsc.subcore_barrier()

    def gather_start(g, b):
      pltpu.async_copy(
          dense_hbm.at[src_v.at[pl.ds(g * CHUNK, CHUNK)]], rows[b], gsem[b])

    def gather_wait(b):
      pltpu.make_async_copy(dense_hbm.at[pl.ds(0, CHUNK)], rows[b],
                            gsem[b]).wait()

    def scatter_wait(b):
      pltpu.make_async_copy(rows[b], acc.at[pl.ds(0, CHUNK)], ssem[b]).wait()

    for j in range(LA):
      gather_start(j, j)

    @pl.loop(0, NCHUNK // NBUF)
    def _block(t):
      for j in range(NBUF):
        g = t * NBUF + j
        gather_wait(j)

        for gg in range(CHUNK // 16):
          ev16 = ev_v[pl.ds(g * CHUNK + gg * 16, 16)]
          for l in range(16):
            s = ev16[l]
            e = gg * 16 + l
            for f in range(D // 16):
              sl = pl.ds(f * 16, 16)
              rows[j][e, sl] = rows[j][e, sl] * s

        pltpu.async_copy(rows[j], acc.at[dst_v.at[g]], ssem[j], add=True)

        nb = (j + LA) % NBUF

        @pl.when(jnp.logical_and(g + LA < NCHUNK, g >= NBUF - LA))
        def _():
          scatter_wait(nb)

        @pl.when(g + LA < NCHUNK)
        def _():
          gather_start(g + LA, nb)

    for j in range(NBUF):
      scatter_wait(j)
    plsc.subcore_barrier()
    pltpu.sync_copy(acc.at[pl.ds(row0, ROWS_PER_SUB)],
                    out_hbm.at[cid, pl.ds(row0, ROWS_PER_SUB)])

  return spmm


CHUNK_HID = 16
CHUNK_CLS = 80
_spmm_hid = _make_spmm(NHID, CHUNK_HID)
_spmm_cls = _make_spmm(NCLASS, CHUNK_CLS)


def _dense1_body(x_ref, w1_ref, v_ref, sup_ref, tilda_ref):
  i = pl.program_id(0)
  s = jnp.dot(x_ref[...], w1_ref[...], preferred_element_type=jnp.float32)
  sup_ref[...] = s
  t = lax.dot_general(v_ref[...], s, (((0,), (0,)), ((), ())),
                      preferred_element_type=jnp.float32)

  @pl.when(i == 0)
  def _():
    tilda_ref[...] = t

  @pl.when(i > 0)
  def _():
    tilda_ref[...] += t


def _dense1(x, w1, v):
  return pl.pallas_call(
      _dense1_body,
      grid=(GRID,),
      in_specs=[
          pl.BlockSpec((RB, NFEAT), lambda i: (i, 0)),
          pl.BlockSpec((NFEAT, NHID), lambda i: (0, 0)),
          pl.BlockSpec((RB, K), lambda i: (i, 0)),
      ],
      out_specs=[
          pl.BlockSpec((RB, NHID), lambda i: (i, 0)),
          pl.BlockSpec((K, NHID), lambda i: (0, 0)),
      ],
      out_shape=[
          jax.ShapeDtypeStruct((N, NHID), jnp.float32),
          jax.ShapeDtypeStruct((K, NHID), jnp.float32),
      ],
  )(x, w1, v)


def _dense2_body(agg_ref, v_ref, tilda_ref, delta_ref, b1_ref, w2_ref,
                 sup2_ref, tilda2_ref):
  i = pl.program_id(0)
  r = jnp.dot(v_ref[...], delta_ref[...] * tilda_ref[...],
              preferred_element_type=jnp.float32)
  h = agg_ref[0] + agg_ref[1] + r + b1_ref[...]
  h = jnp.maximum(h, 0.0)
  s2 = jnp.dot(h, w2_ref[...], preferred_element_type=jnp.float32)
  sup2_ref[...] = s2
  t2 = lax.dot_general(v_ref[...], s2, (((0,), (0,)), ((), ())),
                       preferred_element_type=jnp.float32)

  @pl.when(i == 0)
  def _():
    tilda2_ref[...] = t2

  @pl.when(i > 0)
  def _():
    tilda2_ref[...] += t2


def _dense2(agg, v, tilda, delta, b1, w2):
  return pl.pallas_call(
      _dense2_body,
      grid=(GRID,),
      in_specs=[
          pl.BlockSpec((NC, RB, NHID), lambda i: (0, i, 0)),
          pl.BlockSpec((RB, K), lambda i: (i, 0)),
          pl.BlockSpec((K, NHID), lambda i: (0, 0)),
          pl.BlockSpec((K, 1), lambda i: (0, 0)),
          pl.BlockSpec((1, NHID), lambda i: (0, 0)),
          pl.BlockSpec((NHID, NCLASS), lambda i: (0, 0)),
      ],
      out_specs=[
          pl.BlockSpec((RB, NCLASS), lambda i: (i, 0)),
          pl.BlockSpec((K, NCLASS), lambda i: (0, 0)),
      ],
      out_shape=[
          jax.ShapeDtypeStruct((N, NCLASS), jnp.float32),
          jax.ShapeDtypeStruct((K, NCLASS), jnp.float32),
      ],
  )(agg, v, tilda, delta, b1, w2)


def _dense3_body(agg_ref, v_ref, tilda2_ref, delta_ref, b2_ref, out_ref):
  r = jnp.dot(v_ref[...], delta_ref[...] * tilda2_ref[...],
              preferred_element_type=jnp.float32)
  o = agg_ref[0] + agg_ref[1] + r + b2_ref[...]
  m = jnp.max(o, axis=1, keepdims=True)
  e = o - m
  lse = jnp.log(jnp.sum(jnp.exp(e), axis=1, keepdims=True))
  out_ref[...] = e - lse


def _dense3(agg, v, tilda2, delta, b2):
  return pl.pallas_call(
      _dense3_body,
      grid=(GRID,),
      in_specs=[
          pl.BlockSpec((NC, RB, NCLASS), lambda i: (0, i, 0)),
          pl.BlockSpec((RB, K), lambda i: (i, 0)),
          pl.BlockSpec((K, NCLASS), lambda i: (0, 0)),
          pl.BlockSpec((K, 1), lambda i: (0, 0)),
          pl.BlockSpec((1, NCLASS), lambda i: (0, 0)),
      ],
      out_specs=pl.BlockSpec((RB, NCLASS), lambda i: (i, 0)),
      out_shape=jax.ShapeDtypeStruct((N, NCLASS), jnp.float32),
  )(agg, v, tilda2, delta, b2)


def kernel(x, edge_index, edge_vals, eigvec_mat, gc1_weight, gc1_bias,
           gc2_weight, gc2_bias, delta):
  dst_h = edge_index[0].reshape(NW, EPW // CHUNK_HID, CHUNK_HID)
  dst_c = edge_index[0].reshape(NW, EPW // CHUNK_CLS, CHUNK_CLS)
  src = edge_index[1].reshape(NW, EPW)
  ev = edge_vals.reshape(NW, EPW)
  delta2d = delta[:, None]
  b1 = gc1_bias[None, :]
  b2 = gc2_bias[None, :]

  support, tilda = _dense1(x, gc1_weight, eigvec_mat)
  part1 = _spmm_hid(src, dst_h, ev, support)
  support2, tilda2 = _dense2(part1, eigvec_mat, tilda, delta2d, b1,
                             gc2_weight)
  part2 = _spmm_cls(src, dst_c, ev, support2)
  return _dense3(part2, eigvec_mat, tilda2, delta2d, b2)


# speedup vs baseline: 1.2372x; 1.0279x over previous
import functools

import jax
import jax.numpy as jnp
from jax import lax
from jax.experimental import pallas as pl
from jax.experimental.pallas import tpu as pltpu
from jax.experimental.pallas import tpu_sc as plsc

N = 10000
E = 320000
NFEAT = 128
NHID = 128
NCLASS = 64
K = 32

NC = 2
NS = 16
NW = NC * NS
EPW = E // NW
NPAD = 10240
ROWS_PER_SUB = NPAD // NS

RB = 2000
GRID = N // RB


def _make_spmm(D, CHUNK, NBUF, LA):
  NCHUNK = EPW // CHUNK
  mesh = plsc.VectorSubcoreMesh(core_axis_name="c", subcore_axis_name="s")

  @functools.partial(
      pl.kernel,
      out_type=jax.ShapeDtypeStruct((NC, NPAD, D), jnp.float32),
      mesh=mesh,
      compiler_params=pltpu.CompilerParams(use_tc_tiling_on_sc=False),
      scratch_types=[
          pltpu.VMEM_SHARED((NPAD, D), jnp.float32),
          pltpu.VMEM((EPW,), jnp.int32),
          pltpu.VMEM((NCHUNK, CHUNK), jnp.int32),
          pltpu.VMEM((EPW,), jnp.float32),
          [pltpu.VMEM((CHUNK, D), jnp.float32) for _ in range(NBUF)],
          [pltpu.SemaphoreType.DMA for _ in range(NBUF)],
          [pltpu.SemaphoreType.DMA for _ in range(NBUF)],
      ],
  )
  def spmm(src_hbm, dst_hbm, ev_hbm, dense_hbm, out_hbm,
           acc, src_v, dst_v, ev_v, rows, gsem, ssem):
    cid = lax.axis_index("c")
    sid = lax.axis_index("s")
    wid = sid * NC + cid

    pltpu.async_copy(src_hbm.at[wid], src_v, gsem[0])
    pltpu.async_copy(dst_hbm.at[wid], dst_v, gsem[1])
    pltpu.async_copy(ev_hbm.at[wid], ev_v, gsem[2])

    @pl.loop(0, CHUNK)
    def _zero_row(i):
      for f in range(D // 16):
        rows[0][i, pl.ds(f * 16, 16)] = jnp.zeros((16,), jnp.float32)

    row0 = sid * ROWS_PER_SUB

    for j in range(ROWS_PER_SUB // CHUNK):
      pltpu.async_copy(rows[0], acc.at[pl.ds(row0 + j * CHUNK, CHUNK)],
                       ssem[j % NBUF])
    for j in range(ROWS_PER_SUB // CHUNK):
      pltpu.make_async_copy(rows[0], acc.at[pl.ds(0, CHUNK)],
                            ssem[j % NBUF]).wait()

    pltpu.make_async_copy(src_hbm.at[wid], src_v, gsem[0]).wait()
    pltpu.make_async_copy(dst_hbm.at[wid], dst_v, gsem[1]).wait()
    pltpu.make_async_copy(ev_hbm.at[wid], ev_v, gsem[2]).wait()
    plsc.subcore_barrier()

    def gather_start(g, b):
      pltpu.async_copy(
          dense_hbm.at[src_v.at[pl.ds(g * CHUNK, CHUNK)]], rows[b], gsem[b])

    def gather_wait(b):
      pltpu.make_async_copy(dense_hbm.at[pl.ds(0, CHUNK)], rows[b],
                            gsem[b]).wait()

    def scatter_wait(b):
      pltpu.make_async_copy(rows[b], acc.at[pl.ds(0, CHUNK)], ssem[b]).wait()

    for j in range(LA):
      gather_start(j, j)

    @pl.loop(0, NCHUNK // NBUF)
    def _block(t):
      for j in range(NBUF):
        g = t * NBUF + j
        gather_wait(j)

        for gg in range(CHUNK // 16):
          ev16 = ev_v[pl.ds(g * CHUNK + gg * 16, 16)]
          for l in range(16):
            s = ev16[l]
            e = gg * 16 + l
            for f in range(D // 16):
              sl = pl.ds(f * 16, 16)
              rows[j][e, sl] = rows[j][e, sl] * s

        pltpu.async_copy(rows[j], acc.at[dst_v.at[g]], ssem[j], add=True)

        nb = (j + LA) % NBUF

        @pl.when(jnp.logical_and(g + LA < NCHUNK, g >= NBUF - LA))
        def _():
          scatter_wait(nb)

        @pl.when(g + LA < NCHUNK)
        def _():
          gather_start(g + LA, nb)

    for j in range(NBUF):
      scatter_wait(j)
    plsc.subcore_barrier()
    pltpu.sync_copy(acc.at[pl.ds(row0, ROWS_PER_SUB)],
                    out_hbm.at[cid, pl.ds(row0, ROWS_PER_SUB)])

  return spmm


CHUNK_HID = 16
CHUNK_CLS = 40
_spmm_hid = _make_spmm(NHID, CHUNK_HID, 5, 4)
_spmm_cls = _make_spmm(NCLASS, CHUNK_CLS, 10, 8)


def _dense1_body(x_ref, w1_ref, v_ref, sup_ref, tilda_ref):
  i = pl.program_id(0)
  s = jnp.dot(x_ref[...], w1_ref[...], preferred_element_type=jnp.float32)
  sup_ref[...] = s
  t = lax.dot_general(v_ref[...], s, (((0,), (0,)), ((), ())),
                      preferred_element_type=jnp.float32)

  @pl.when(i == 0)
  def _():
    tilda_ref[...] = t

  @pl.when(i > 0)
  def _():
    tilda_ref[...] += t


def _dense1(x, w1, v):
  return pl.pallas_call(
      _dense1_body,
      grid=(GRID,),
      in_specs=[
          pl.BlockSpec((RB, NFEAT), lambda i: (i, 0)),
          pl.BlockSpec((NFEAT, NHID), lambda i: (0, 0)),
          pl.BlockSpec((RB, K), lambda i: (i, 0)),
      ],
      out_specs=[
          pl.BlockSpec((RB, NHID), lambda i: (i, 0)),
          pl.BlockSpec((K, NHID), lambda i: (0, 0)),
      ],
      out_shape=[
          jax.ShapeDtypeStruct((N, NHID), jnp.float32),
          jax.ShapeDtypeStruct((K, NHID), jnp.float32),
      ],
  )(x, w1, v)


def _dense2_body(agg_ref, v_ref, tilda_ref, delta_ref, b1_ref, w2_ref,
                 sup2_ref, tilda2_ref):
  i = pl.program_id(0)
  r = jnp.dot(v_ref[...], delta_ref[...] * tilda_ref[...],
              preferred_element_type=jnp.float32)
  h = agg_ref[0] + agg_ref[1] + r + b1_ref[...]
  h = jnp.maximum(h, 0.0)
  s2 = jnp.dot(h, w2_ref[...], preferred_element_type=jnp.float32)
  sup2_ref[...] = s2
  t2 = lax.dot_general(v_ref[...], s2, (((0,), (0,)), ((), ())),
                       preferred_element_type=jnp.float32)

  @pl.when(i == 0)
  def _():
    tilda2_ref[...] = t2

  @pl.when(i > 0)
  def _():
    tilda2_ref[...] += t2


def _dense2(agg, v, tilda, delta, b1, w2):
  return pl.pallas_call(
      _dense2_body,
      grid=(GRID,),
      in_specs=[
          pl.BlockSpec((NC, RB, NHID), lambda i: (0, i, 0)),
          pl.BlockSpec((RB, K), lambda i: (i, 0)),
          pl.BlockSpec((K, NHID), lambda i: (0, 0)),
          pl.BlockSpec((K, 1), lambda i: (0, 0)),
          pl.BlockSpec((1, NHID), lambda i: (0, 0)),
          pl.BlockSpec((NHID, NCLASS), lambda i: (0, 0)),
      ],
      out_specs=[
          pl.BlockSpec((RB, NCLASS), lambda i: (i, 0)),
          pl.BlockSpec((K, NCLASS), lambda i: (0, 0)),
      ],
      out_shape=[
          jax.ShapeDtypeStruct((N, NCLASS), jnp.float32),
          jax.ShapeDtypeStruct((K, NCLASS), jnp.float32),
      ],
  )(agg, v, tilda, delta, b1, w2)


def _dense3_body(agg_ref, v_ref, tilda2_ref, delta_ref, b2_ref, out_ref):
  r = jnp.dot(v_ref[...], delta_ref[...] * tilda2_ref[...],
              preferred_element_type=jnp.float32)
  o = agg_ref[0] + agg_ref[1] + r + b2_ref[...]
  m = jnp.max(o, axis=1, keepdims=True)
  e = o - m
  lse = jnp.log(jnp.sum(jnp.exp(e), axis=1, keepdims=True))
  out_ref[...] = e - lse


def _dense3(agg, v, tilda2, delta, b2):
  return pl.pallas_call(
      _dense3_body,
      grid=(GRID,),
      in_specs=[
          pl.BlockSpec((NC, RB, NCLASS), lambda i: (0, i, 0)),
          pl.BlockSpec((RB, K), lambda i: (i, 0)),
          pl.BlockSpec((K, NCLASS), lambda i: (0, 0)),
          pl.BlockSpec((K, 1), lambda i: (0, 0)),
          pl.BlockSpec((1, NCLASS), lambda i: (0, 0)),
      ],
      out_specs=pl.BlockSpec((RB, NCLASS), lambda i: (i, 0)),
      out_shape=jax.ShapeDtypeStruct((N, NCLASS), jnp.float32),
  )(agg, v, tilda2, delta, b2)


def kernel(x, edge_index, edge_vals, eigvec_mat, gc1_weight, gc1_bias,
           gc2_weight, gc2_bias, delta):
  dst_h = edge_index[0].reshape(NW, EPW // CHUNK_HID, CHUNK_HID)
  dst_c = edge_index[0].reshape(NW, EPW // CHUNK_CLS, CHUNK_CLS)
  src = edge_index[1].reshape(NW, EPW)
  ev = edge_vals.reshape(NW, EPW)
  delta2d = delta[:, None]
  b1 = gc1_bias[None, :]
  b2 = gc2_bias[None, :]

  support, tilda = _dense1(x, gc1_weight, eigvec_mat)
  part1 = _spmm_hid(src, dst_h, ev, support)
  support2, tilda2 = _dense2(part1, eigvec_mat, tilda, delta2d, b1,
                             gc2_weight)
  part2 = _spmm_cls(src, dst_c, ev, support2)
  return _dense3(part2, eigvec_mat, tilda2, delta2d, b2)
